# Initial kernel scaffold; baseline (speedup 1.0000x reference)
import jax
import jax.numpy as jnp
from jax.experimental import pallas as pl


def kernel(x, edge_index, W1l, b1l, W1r, g1, be1, W2l, b2l, W2r):
    raise NotImplementedError("write your pallas kernel here")


# speedup vs baseline: 5.8452x; 5.8452x over previous
import functools

import jax
import jax.numpy as jnp
from jax import lax
from jax.experimental import pallas as pl
from jax.experimental.pallas import tpu as pltpu
from jax.experimental.pallas import tpu_sc as plsc

_N = 10000
_D = 128
_E = 320000
_NC = 2
_NS = 16
_NW = _NC * _NS
_BLK = 64
_NBLK = _E // _BLK
_BASE = _NBLK // _NW
_XTRA = _NBLK - _BASE * _NW
_NRING = 2
_DRING = 4
_RPW = 632
_RPW_LAST = _N - (_NS - 1) * _RPW


def _worker_id():
  return lax.axis_index("s") * _NC + lax.axis_index("c")


def _zero_fill(buf, nrows):
  zero16 = jnp.zeros((16,), jnp.float32)

  def zrow(i, carry):
    for j in range(_D // 16):
      buf[i, pl.ds(j * 16, 16)] = zero16
    return carry

  lax.fori_loop(0, nrows, zrow, 0)


def _zero_spmem(acc_sh, zbuf, base, nrows):
  for t in range(nrows // _BLK):
    pltpu.sync_copy(zbuf, acc_sh.at[pl.ds(base + t * _BLK, _BLK)])
  rem = nrows - (nrows // _BLK) * _BLK
  if rem:
    pltpu.sync_copy(zbuf.at[pl.ds(0, rem)],
                    acc_sh.at[pl.ds(base + nrows - rem, rem)])


def _make_segsum():
  mesh = plsc.VectorSubcoreMesh(core_axis_name="c", subcore_axis_name="s")

  @functools.partial(
      pl.kernel,
      out_type=jax.ShapeDtypeStruct((_NC, _N, _D), jnp.float32),
      mesh=mesh,
      scratch_types=[
          pltpu.VMEM((_NRING, _BLK), jnp.int32),
          pltpu.VMEM((_NRING, _BLK), jnp.int32),
          pltpu.VMEM((_NRING, _BLK, _D), jnp.float32),
          pltpu.VMEM_SHARED((_N, _D), jnp.float32),
          pltpu.SemaphoreType.DMA,
          pltpu.SemaphoreType.DMA,
      ],
  )
  def seg(table_h, src_h, dst_h, acc_out, src_v, dst_v, rows_v, acc_sh,
          sem0, sem1):
    sems = (sem0, sem1)
    c = lax.axis_index("c")
    s = lax.axis_index("s")
    wid = _worker_id()

    _zero_fill(rows_v.at[0], _BLK)
    base = s * _RPW

    @pl.when(s < _NS - 1)
    def _():
      _zero_spmem(acc_sh, rows_v.at[0], base, _RPW)

    @pl.when(s == _NS - 1)
    def _():
      _zero_spmem(acc_sh, rows_v.at[0], base, _RPW_LAST)

    plsc.subcore_barrier()

    start_blk = wid * _BASE + jnp.minimum(wid, _XTRA)

    def fire(blk, j):
      off = blk * _BLK
      pltpu.sync_copy(src_h.at[pl.ds(off, _BLK)], src_v.at[j])
      pltpu.sync_copy(dst_h.at[pl.ds(off, _BLK)], dst_v.at[j])
      return pltpu.async_copy(table_h.at[src_v.at[j]], rows_v.at[j], sems[j])

    def drain(cp, j):
      cp.wait()
      pltpu.sync_copy(rows_v.at[j], acc_sh.at[dst_v.at[j]], add=True)

    def group(g, carry):
      b0 = start_blk + g * _NRING
      cps = [fire(b0 + j, j) for j in range(_NRING)]
      for j in range(_NRING):
        drain(cps[j], j)
      return carry

    lax.fori_loop(0, _BASE // _NRING, group, 0)

    @pl.when(wid < _XTRA)
    def _():
      drain(fire(start_blk + _BASE, 0), 0)

    plsc.subcore_barrier()

    @pl.when(s < _NS - 1)
    def _():
      pltpu.sync_copy(acc_sh.at[pl.ds(base, _RPW)],
                      acc_out.at[c, pl.ds(base, _RPW)])

    @pl.when(s == _NS - 1)
    def _():
      pltpu.sync_copy(acc_sh.at[pl.ds(base, _RPW_LAST)],
                      acc_out.at[c, pl.ds(base, _RPW_LAST)])

  return seg


def _make_deg():
  mesh = plsc.VectorSubcoreMesh(core_axis_name="c", subcore_axis_name="s")

  @functools.partial(
      pl.kernel,
      out_type=jax.ShapeDtypeStruct((_NC, _N, _D), jnp.float32),
      mesh=mesh,
      scratch_types=[
          pltpu.VMEM((_DRING, _BLK), jnp.int32),
          pltpu.VMEM((_BLK, _D), jnp.float32),
          pltpu.VMEM((_BLK, _D), jnp.float32),
          pltpu.VMEM_SHARED((_N, _D), jnp.float32),
      ] + [pltpu.SemaphoreType.DMA] * _DRING,
  )
  def deg(dst_h, deg_out, dst_v, ones_v, zros_v, acc_sh, *sems):
    c = lax.axis_index("c")
    s = lax.axis_index("s")
    wid = _worker_id()

    one16 = jnp.ones((16,), jnp.float32)

    def orow(i, carry):
      for j in range(_D // 16):
        ones_v[i, pl.ds(j * 16, 16)] = one16
      return carry

    lax.fori_loop(0, _BLK, orow, 0)
    _zero_fill(zros_v, _BLK)
    base = s * _RPW

    @pl.when(s < _NS - 1)
    def _():
      _zero_spmem(acc_sh, zros_v, base, _RPW)

    @pl.when(s == _NS - 1)
    def _():
      _zero_spmem(acc_sh, zros_v, base, _RPW_LAST)

    plsc.subcore_barrier()

    start_blk = wid * _BASE + jnp.minimum(wid, _XTRA)

    def fire(blk, j):
      off = blk * _BLK
      pltpu.sync_copy(dst_h.at[pl.ds(off, _BLK)], dst_v.at[j])
      return pltpu.async_copy(ones_v, acc_sh.at[dst_v.at[j]], sems[j],
                              add=True)

    def group(g, carry):
      b0 = start_blk + g * _DRING
      cps = [fire(b0 + j, j) for j in range(_DRING)]
      for cp in cps:
        cp.wait()
      return carry

    lax.fori_loop(0, _BASE // _DRING, group, 0)

    @pl.when(wid < _XTRA)
    def _():
      fire(start_blk + _BASE, 0).wait()

    plsc.subcore_barrier()

    @pl.when(s < _NS - 1)
    def _():
      pltpu.sync_copy(acc_sh.at[pl.ds(base, _RPW)],
                      deg_out.at[c, pl.ds(base, _RPW)])

    @pl.when(s == _NS - 1)
    def _():
      pltpu.sync_copy(acc_sh.at[pl.ds(base, _RPW_LAST)],
                      deg_out.at[c, pl.ds(base, _RPW_LAST)])

  return deg


@functools.lru_cache(maxsize=None)
def _get_segsum():
  return _make_segsum()


@functools.lru_cache(maxsize=None)
def _get_deg():
  return _make_deg()


_R = 1000


def _dense1_body(accp, degp, x, wl, wr, b, g, be, h_out, inv_out):
  ssum = accp[0] + accp[1]
  dg = (degp[0] + degp[1])[:, 0:1]
  invd = 1.0 / jnp.maximum(dg, 1.0)
  mean = ssum * invd
  h = (jnp.dot(mean, wl[...], preferred_element_type=jnp.float32)
       + jnp.dot(x[...], wr[...], preferred_element_type=jnp.float32)
       + b[...])
  mu = jnp.mean(h, axis=1, keepdims=True)
  var = jnp.mean((h - mu) ** 2, axis=1, keepdims=True)
  h = (h - mu) * lax.rsqrt(var + 1e-5) * g[...] + be[...]
  h_out[...] = jnp.maximum(h, 0.0)
  inv_out[...] = jnp.broadcast_to(invd, (_R, _D))


def _dense1(acc1, deg1, x, wlT, wrT, b, g, be):
  grid = (_N // _R,)
  return pl.pallas_call(
      lambda a, d, xx, wl, wr, bb, gg, bbe, ho, io: _dense1_body(
          a[...], d[...], xx, wl, wr, bb, gg, bbe, ho, io),
      grid=grid,
      in_specs=[
          pl.BlockSpec((_NC, _R, _D), lambda i: (0, i, 0)),
          pl.BlockSpec((_NC, _R, _D), lambda i: (0, i, 0)),
          pl.BlockSpec((_R, _D), lambda i: (i, 0)),
          pl.BlockSpec((_D, _D), lambda i: (0, 0)),
          pl.BlockSpec((_D, _D), lambda i: (0, 0)),
          pl.BlockSpec((1, _D), lambda i: (0, 0)),
          pl.BlockSpec((1, _D), lambda i: (0, 0)),
          pl.BlockSpec((1, _D), lambda i: (0, 0)),
      ],
      out_specs=[
          pl.BlockSpec((_R, _D), lambda i: (i, 0)),
          pl.BlockSpec((_R, _D), lambda i: (i, 0)),
      ],
      out_shape=[
          jax.ShapeDtypeStruct((_N, _D), jnp.float32),
          jax.ShapeDtypeStruct((_N, _D), jnp.float32),
      ],
  )(acc1, deg1, x, wlT, wrT, b, g, be)


def _dense2_body(accp, invd, h, wl, wr, b, out):
  mean = (accp[0] + accp[1]) * invd[...]
  o = (jnp.dot(mean, wl[...], preferred_element_type=jnp.float32)
       + jnp.dot(h[...], wr[...], preferred_element_type=jnp.float32)
       + b[...])
  nrm = jnp.sqrt(jnp.sum(o * o, axis=1, keepdims=True))
  out[...] = o / jnp.maximum(nrm, 1e-12)


def _dense2(acc2, invd, h, wlT, wrT, b):
  grid = (_N // _R,)
  return pl.pallas_call(
      lambda a, iv, hh, wl, wr, bb, oo: _dense2_body(
          a[...], iv, hh, wl, wr, bb, oo),
      grid=grid,
      in_specs=[
          pl.BlockSpec((_NC, _R, _D), lambda i: (0, i, 0)),
          pl.BlockSpec((_R, _D), lambda i: (i, 0)),
          pl.BlockSpec((_R, _D), lambda i: (i, 0)),
          pl.BlockSpec((_D, _D), lambda i: (0, 0)),
          pl.BlockSpec((_D, _D), lambda i: (0, 0)),
          pl.BlockSpec((1, _D), lambda i: (0, 0)),
      ],
      out_specs=pl.BlockSpec((_R, _D), lambda i: (i, 0)),
      out_shape=jax.ShapeDtypeStruct((_N, _D), jnp.float32),
  )(acc2, invd, h, wlT, wrT, b)


def kernel(x, edge_index, W1l, b1l, W1r, g1, be1, W2l, b2l, W2r):
  src = edge_index[0]
  dst = edge_index[1]
  acc1 = _get_segsum()(x, src, dst)
  deg1 = _get_deg()(dst)
  h, invd = _dense1(acc1, deg1, x, W1l.T, W1r.T,
                    b1l.reshape(1, _D), g1.reshape(1, _D),
                    be1.reshape(1, _D))
  acc2 = _get_segsum()(h, src, dst)
  return _dense2(acc2, invd, h, W2l.T, W2r.T, b2l.reshape(1, _D))
